# baseline (speedup 1.0000x reference)
import functools

import jax
import jax.numpy as jnp
from jax import lax
from jax.experimental import pallas as pl
from jax.experimental.pallas import tpu as pltpu
from jax.experimental.pallas import tpu_sc as plsc

_NC = 2
_NS = 16


def _to_linear_rows(emb):
    V, D = emb.shape
    embT = emb.T
    BN = 8192
    grid = ((V + BN - 1) // BN,)

    def body(in_ref, o_ref):
        tt = in_ref[...].T
        o_ref[...] = jnp.concatenate([tt, tt], axis=1)

    return pl.pallas_call(
        body,
        grid=grid,
        in_specs=[pl.BlockSpec((D, BN), lambda i: (0, i))],
        out_specs=pl.BlockSpec((BN, 2 * D), lambda i: (i, 0)),
        out_shape=jax.ShapeDtypeStruct((V, 2 * D), jnp.float32),
    )(embT)


def _gather_concat(idx, emb):
    F, B = idx.shape
    V, D = emb.shape
    nw = _NC * _NS
    bpw = B // nw

    mesh = plsc.VectorSubcoreMesh(core_axis_name="c", subcore_axis_name="s")

    @functools.partial(
        pl.kernel,
        out_type=jax.ShapeDtypeStruct((F // 2, B, 2 * D), emb.dtype),
        mesh=mesh,
        scratch_types=[
            pltpu.VMEM((F, bpw), jnp.int32),
            pltpu.VMEM((bpw, D), emb.dtype),
            pltpu.VMEM((bpw, D), emb.dtype),
            pltpu.SemaphoreType.DMA,
            pltpu.SemaphoreType.DMA,
        ],
        compiler_params=pltpu.CompilerParams(use_tc_tiling_on_sc=False),
    )
    def gk(idx_hbm, emb_hbm, out_hbm, idx_v, buf0, buf1, sem0, sem1):
        wid = lax.axis_index("s") * _NC + lax.axis_index("c")
        base = wid * bpw
        pltpu.sync_copy(idx_hbm.at[:, pl.ds(base, bpw)], idx_v)
        bufs = (buf0, buf1)
        sems = (sem0, sem1)
        cp = pltpu.async_copy(emb_hbm.at[idx_v.at[0]], bufs[0], sems[0])
        for f in range(F):
            cur = f % 2
            nxt = 1 - cur
            cp_next = None
            if f + 1 < F:
                cp_next = pltpu.async_copy(
                    emb_hbm.at[idx_v.at[f + 1]], bufs[nxt], sems[nxt]
                )
            cp.wait()
            pltpu.sync_copy(
                bufs[cur],
                out_hbm.at[f // 2, pl.ds(base, bpw), pl.ds((f % 2) * D, D)],
            )
            cp = cp_next

    return gk(idx, emb)


def _mlp(x3, W0, b0, W1, b1, W2, b2, W3, b3, block_m=1024):
    S, B, C = x3.shape

    def body(x_ref, w0_ref, b0_ref, w1_ref, b1_ref, w2_ref, b2_ref,
             w3_ref, b3_ref, o_ref):
        f32 = jnp.float32
        x = x_ref[...].astype(jnp.bfloat16)
        ps = [jnp.dot(x[c], w0_ref[c], preferred_element_type=f32)
              for c in range(S)]
        while len(ps) > 1:
            ps = [a + b for a, b in zip(ps[::2], ps[1::2])] + (
                [ps[-1]] if len(ps) % 2 else [])
        h = ps[0] + b0_ref[...]
        h = jnp.maximum(h, 0.0).astype(jnp.bfloat16)
        h = jnp.dot(h, w1_ref[...], preferred_element_type=f32) + b1_ref[...]
        h = jnp.maximum(h, 0.0).astype(jnp.bfloat16)
        h = jnp.dot(h, w2_ref[...], preferred_element_type=f32) + b2_ref[...]
        h = jnp.maximum(h, 0.0).astype(jnp.bfloat16)
        o_ref[...] = jnp.dot(h, w3_ref[...], preferred_element_type=f32) + b3_ref[...]

    def _full(w):
        return pl.BlockSpec(w.shape, lambda i: (0,) * w.ndim)

    bf16 = jnp.bfloat16
    w03 = W0.reshape(S, C, W0.shape[1]).astype(bf16)
    W1, W2, W3 = W1.astype(bf16), W2.astype(bf16), W3.astype(bf16)
    return pl.pallas_call(
        body,
        grid=(B // block_m,),
        in_specs=[
            pl.BlockSpec((S, block_m, C), lambda i: (0, i, 0)),
            _full(w03), _full(b0), _full(W1), _full(b1),
            _full(W2), _full(b2), _full(W3), _full(b3),
        ],
        out_specs=pl.BlockSpec((block_m, 1), lambda i: (i, 0)),
        out_shape=jax.ShapeDtypeStruct((B, 1), jnp.float32),
    )(x3, w03, b0, W1, b1, W2, b2, W3, b3)


def kernel(idx, emb, W0, b0, W1, b1, W2, b2, W3, b3):
    idx = idx.astype(jnp.int32) * 2
    emb_lin = _to_linear_rows(emb).reshape(2 * emb.shape[0], emb.shape[1])
    B = idx.shape[1]
    nh = 2
    bh = B // nh
    biases = (b0.reshape(1, -1), b1.reshape(1, -1),
              b2.reshape(1, -1), b3.reshape(1, -1))
    xs = [_gather_concat(idx[:, k * bh:(k + 1) * bh], emb_lin)
          for k in range(nh)]
    outs = [_mlp(x3, W0, biases[0], W1, biases[1], W2, biases[2],
                 W3, biases[3], block_m=bh // 2)
            for x3 in xs]
    return jnp.concatenate(outs, axis=0)


# speedup vs baseline: 1.0152x; 1.0152x over previous
import functools

import jax
import jax.numpy as jnp
from jax import lax
from jax.experimental import pallas as pl
from jax.experimental.pallas import tpu as pltpu
from jax.experimental.pallas import tpu_sc as plsc

_NC = 2
_NS = 16


def _to_linear_rows(emb):
    V, D = emb.shape
    embT = emb.T
    BN = 12800
    grid = ((V + BN - 1) // BN,)

    def body(in_ref, o_ref):
        tt = in_ref[...].T
        o_ref[...] = jnp.concatenate([tt, tt], axis=1)

    return pl.pallas_call(
        body,
        grid=grid,
        in_specs=[pl.BlockSpec((D, BN), lambda i: (0, i))],
        out_specs=pl.BlockSpec((BN, 2 * D), lambda i: (i, 0)),
        out_shape=jax.ShapeDtypeStruct((V, 2 * D), jnp.float32),
    )(embT)


def _gather_concat(idx, emb):
    F, B = idx.shape
    V, D = emb.shape
    nw = _NC * _NS
    bpw = B // nw

    mesh = plsc.VectorSubcoreMesh(core_axis_name="c", subcore_axis_name="s")

    @functools.partial(
        pl.kernel,
        out_type=jax.ShapeDtypeStruct((F // 2, B, 2 * D), emb.dtype),
        mesh=mesh,
        scratch_types=[
            pltpu.VMEM((F, bpw), jnp.int32),
            pltpu.VMEM((bpw, D), emb.dtype),
            pltpu.VMEM((bpw, D), emb.dtype),
            pltpu.SemaphoreType.DMA,
            pltpu.SemaphoreType.DMA,
        ],
        compiler_params=pltpu.CompilerParams(use_tc_tiling_on_sc=False),
    )
    def gk(idx_hbm, emb_hbm, out_hbm, idx_v, buf0, buf1, sem0, sem1):
        wid = lax.axis_index("s") * _NC + lax.axis_index("c")
        base = wid * bpw
        pltpu.sync_copy(idx_hbm.at[:, pl.ds(base, bpw)], idx_v)
        bufs = (buf0, buf1)
        sems = (sem0, sem1)
        cp = pltpu.async_copy(emb_hbm.at[idx_v.at[0]], bufs[0], sems[0])
        for f in range(F):
            cur = f % 2
            nxt = 1 - cur
            cp_next = None
            if f + 1 < F:
                cp_next = pltpu.async_copy(
                    emb_hbm.at[idx_v.at[f + 1]], bufs[nxt], sems[nxt]
                )
            cp.wait()
            pltpu.sync_copy(
                bufs[cur],
                out_hbm.at[f // 2, pl.ds(base, bpw), pl.ds((f % 2) * D, D)],
            )
            cp = cp_next

    return gk(idx, emb)


def _mlp(x3, W0, b0, W1, b1, W2, b2, W3, b3, block_m=1024):
    S, B, C = x3.shape

    def body(x_ref, w0_ref, b0_ref, w1_ref, b1_ref, w2_ref, b2_ref,
             w3_ref, b3_ref, o_ref):
        f32 = jnp.float32
        x = x_ref[...].astype(jnp.bfloat16)
        ps = [jnp.dot(x[c], w0_ref[c], preferred_element_type=f32)
              for c in range(S)]
        while len(ps) > 1:
            ps = [a + b for a, b in zip(ps[::2], ps[1::2])] + (
                [ps[-1]] if len(ps) % 2 else [])
        h = ps[0] + b0_ref[...]
        h = jnp.maximum(h, 0.0).astype(jnp.bfloat16)
        h = jnp.dot(h, w1_ref[...], preferred_element_type=f32) + b1_ref[...]
        h = jnp.maximum(h, 0.0).astype(jnp.bfloat16)
        h = jnp.dot(h, w2_ref[...], preferred_element_type=f32) + b2_ref[...]
        h = jnp.maximum(h, 0.0).astype(jnp.bfloat16)
        o_ref[...] = jnp.dot(h, w3_ref[...], preferred_element_type=f32) + b3_ref[...]

    def _full(w):
        return pl.BlockSpec(w.shape, lambda i: (0,) * w.ndim)

    bf16 = jnp.bfloat16
    w03 = W0.reshape(S, C, W0.shape[1]).astype(bf16)
    W1, W2, W3 = W1.astype(bf16), W2.astype(bf16), W3.astype(bf16)
    return pl.pallas_call(
        body,
        grid=(B // block_m,),
        in_specs=[
            pl.BlockSpec((S, block_m, C), lambda i: (0, i, 0)),
            _full(w03), _full(b0), _full(W1), _full(b1),
            _full(W2), _full(b2), _full(W3), _full(b3),
        ],
        out_specs=pl.BlockSpec((block_m, 1), lambda i: (i, 0)),
        out_shape=jax.ShapeDtypeStruct((B, 1), jnp.float32),
    )(x3, w03, b0, W1, b1, W2, b2, W3, b3)


def kernel(idx, emb, W0, b0, W1, b1, W2, b2, W3, b3):
    idx = idx.astype(jnp.int32) * 2
    emb_lin = _to_linear_rows(emb).reshape(2 * emb.shape[0], emb.shape[1])
    B = idx.shape[1]
    nh = 2
    bh = B // nh
    biases = (b0.reshape(1, -1), b1.reshape(1, -1),
              b2.reshape(1, -1), b3.reshape(1, -1))
    xs = [_gather_concat(idx[:, k * bh:(k + 1) * bh], emb_lin)
          for k in range(nh)]
    outs = [_mlp(x3, W0, biases[0], W1, biases[1], W2, biases[2],
                 W3, biases[3], block_m=bh // 4)
            for x3 in xs]
    return jnp.concatenate(outs, axis=0)


# speedup vs baseline: 1.0524x; 1.0367x over previous
import functools

import jax
import jax.numpy as jnp
from jax import lax
from jax.experimental import pallas as pl
from jax.experimental.pallas import tpu as pltpu
from jax.experimental.pallas import tpu_sc as plsc

_NC = 2
_NS = 16


def _to_linear_rows(emb):
    V, D = emb.shape
    embT = emb.T
    BN = 12800
    grid = ((V + BN - 1) // BN,)

    def body(in_ref, o_ref):
        tt = in_ref[...].T
        o_ref[...] = jnp.concatenate([tt, tt], axis=1)

    return pl.pallas_call(
        body,
        grid=grid,
        in_specs=[pl.BlockSpec((D, BN), lambda i: (0, i))],
        out_specs=pl.BlockSpec((BN, 2 * D), lambda i: (i, 0)),
        out_shape=jax.ShapeDtypeStruct((V, 2 * D), jnp.float32),
    )(embT)


def _gather_concat(idx, emb):
    F, B = idx.shape
    V, D = emb.shape
    nw = _NC * _NS
    bpw = B // nw

    mesh = plsc.VectorSubcoreMesh(core_axis_name="c", subcore_axis_name="s")

    K = 3

    @functools.partial(
        pl.kernel,
        out_type=jax.ShapeDtypeStruct((F // 2, B, 2 * D), emb.dtype),
        mesh=mesh,
        scratch_types=[
            pltpu.VMEM((F, bpw), jnp.int32),
        ] + [pltpu.VMEM((bpw, D), emb.dtype) for _ in range(K)]
          + [pltpu.SemaphoreType.DMA for _ in range(2 * K)],
        compiler_params=pltpu.CompilerParams(use_tc_tiling_on_sc=False),
    )
    def gk(idx_hbm, emb_hbm, out_hbm, idx_v, *rest):
        bufs = rest[:K]
        gsem = rest[K:2 * K]
        ssem = rest[2 * K:3 * K]
        wid = lax.axis_index("s") * _NC + lax.axis_index("c")
        base = wid * bpw
        pltpu.sync_copy(idx_hbm.at[:, pl.ds(base, bpw)], idx_v)
        gcp = [None] * K
        scp = [None] * K

        def _store(fd):
            kd = fd % K
            gcp[kd].wait()
            scp[kd] = pltpu.async_copy(
                bufs[kd],
                out_hbm.at[fd // 2, pl.ds(base, bpw), pl.ds((fd % 2) * D, D)],
                ssem[kd],
            )

        for f in range(F):
            k = f % K
            if scp[k] is not None:
                scp[k].wait()
            gcp[k] = pltpu.async_copy(emb_hbm.at[idx_v.at[f]], bufs[k], gsem[k])
            if f - (K - 1) >= 0:
                _store(f - (K - 1))
        for fd in range(max(F - (K - 1), 0), F):
            _store(fd)
        for k in range(K):
            if scp[k] is not None:
                scp[k].wait()

    return gk(idx, emb)


def _mlp(x3, W0, b0, W1, b1, W2, b2, W3, b3, block_m=1024):
    S, B, C = x3.shape

    def body(x_ref, w0_ref, b0_ref, w1_ref, b1_ref, w2_ref, b2_ref,
             w3_ref, b3_ref, o_ref):
        f32 = jnp.float32
        x = x_ref[...].astype(jnp.bfloat16)
        ps = [jnp.dot(x[c], w0_ref[c], preferred_element_type=f32)
              for c in range(S)]
        while len(ps) > 1:
            ps = [a + b for a, b in zip(ps[::2], ps[1::2])] + (
                [ps[-1]] if len(ps) % 2 else [])
        h = ps[0] + b0_ref[...]
        h = jnp.maximum(h, 0.0).astype(jnp.bfloat16)
        h = jnp.dot(h, w1_ref[...], preferred_element_type=f32) + b1_ref[...]
        h = jnp.maximum(h, 0.0).astype(jnp.bfloat16)
        h = jnp.dot(h, w2_ref[...], preferred_element_type=f32) + b2_ref[...]
        h = jnp.maximum(h, 0.0).astype(jnp.bfloat16)
        o_ref[...] = jnp.dot(h, w3_ref[...], preferred_element_type=f32) + b3_ref[...]

    def _full(w):
        return pl.BlockSpec(w.shape, lambda i: (0,) * w.ndim)

    bf16 = jnp.bfloat16
    w03 = W0.reshape(S, C, W0.shape[1]).astype(bf16)
    W1, W2, W3 = W1.astype(bf16), W2.astype(bf16), W3.astype(bf16)
    return pl.pallas_call(
        body,
        grid=(B // block_m,),
        in_specs=[
            pl.BlockSpec((S, block_m, C), lambda i: (0, i, 0)),
            _full(w03), _full(b0), _full(W1), _full(b1),
            _full(W2), _full(b2), _full(W3), _full(b3),
        ],
        out_specs=pl.BlockSpec((block_m, 1), lambda i: (i, 0)),
        out_shape=jax.ShapeDtypeStruct((B, 1), jnp.float32),
    )(x3, w03, b0, W1, b1, W2, b2, W3, b3)


def kernel(idx, emb, W0, b0, W1, b1, W2, b2, W3, b3):
    idx = idx.astype(jnp.int32) * 2
    emb_lin = _to_linear_rows(emb).reshape(2 * emb.shape[0], emb.shape[1])
    B = idx.shape[1]
    nh = 2
    bh = B // nh
    biases = (b0.reshape(1, -1), b1.reshape(1, -1),
              b2.reshape(1, -1), b3.reshape(1, -1))
    xs = [_gather_concat(idx[:, k * bh:(k + 1) * bh], emb_lin)
          for k in range(nh)]
    outs = [_mlp(x3, W0, biases[0], W1, biases[1], W2, biases[2],
                 W3, biases[3], block_m=bh // 4)
            for x3 in xs]
    return jnp.concatenate(outs, axis=0)


# speedup vs baseline: 1.0665x; 1.0134x over previous
import functools

import jax
import jax.numpy as jnp
from jax import lax
from jax.experimental import pallas as pl
from jax.experimental.pallas import tpu as pltpu
from jax.experimental.pallas import tpu_sc as plsc

_NC = 2
_NS = 16


def _to_linear_rows(emb):
    V, D = emb.shape
    embT = emb.T
    BN = 12800
    grid = ((V + BN - 1) // BN,)

    def body(in_ref, o_ref):
        tt = in_ref[...].T
        o_ref[...] = jnp.concatenate([tt, tt], axis=1)

    return pl.pallas_call(
        body,
        grid=grid,
        in_specs=[pl.BlockSpec((D, BN), lambda i: (0, i))],
        out_specs=pl.BlockSpec((BN, 2 * D), lambda i: (i, 0)),
        out_shape=jax.ShapeDtypeStruct((V, 2 * D), jnp.float32),
    )(embT)


def _gather_concat(idx, emb):
    F, B = idx.shape
    V, D = emb.shape
    nw = _NC * _NS
    bpw = B // nw

    mesh = plsc.VectorSubcoreMesh(core_axis_name="c", subcore_axis_name="s")

    K = 5

    @functools.partial(
        pl.kernel,
        out_type=jax.ShapeDtypeStruct((F // 2, B, 2 * D), emb.dtype),
        mesh=mesh,
        scratch_types=[
            pltpu.VMEM((F, bpw), jnp.int32),
        ] + [pltpu.VMEM((bpw, D), emb.dtype) for _ in range(K)]
          + [pltpu.SemaphoreType.DMA for _ in range(2 * K)],
        compiler_params=pltpu.CompilerParams(use_tc_tiling_on_sc=False),
    )
    def gk(idx_hbm, emb_hbm, out_hbm, idx_v, *rest):
        bufs = rest[:K]
        gsem = rest[K:2 * K]
        ssem = rest[2 * K:3 * K]
        wid = lax.axis_index("s") * _NC + lax.axis_index("c")
        base = wid * bpw
        pltpu.sync_copy(idx_hbm.at[:, pl.ds(base, bpw)], idx_v)
        gcp = [None] * K
        scp = [None] * K

        def _store(fd):
            kd = fd % K
            gcp[kd].wait()
            scp[kd] = pltpu.async_copy(
                bufs[kd],
                out_hbm.at[fd // 2, pl.ds(base, bpw), pl.ds((fd % 2) * D, D)],
                ssem[kd],
            )

        for f in range(F):
            k = f % K
            if scp[k] is not None:
                scp[k].wait()
            gcp[k] = pltpu.async_copy(emb_hbm.at[idx_v.at[f]], bufs[k], gsem[k])
            if f - (K - 1) >= 0:
                _store(f - (K - 1))
        for fd in range(max(F - (K - 1), 0), F):
            _store(fd)
        for k in range(K):
            if scp[k] is not None:
                scp[k].wait()

    return gk(idx, emb)


def _mlp(x3, W0, b0, W1, b1, W2, b2, W3, b3, block_m=1024):
    S, B, C = x3.shape

    def body(x_ref, w0_ref, b0_ref, w1_ref, b1_ref, w2_ref, b2_ref,
             w3_ref, b3_ref, o_ref):
        f32 = jnp.float32
        x = x_ref[...].astype(jnp.bfloat16)
        ps = [jnp.dot(x[c], w0_ref[c], preferred_element_type=f32)
              for c in range(S)]
        while len(ps) > 1:
            ps = [a + b for a, b in zip(ps[::2], ps[1::2])] + (
                [ps[-1]] if len(ps) % 2 else [])
        h = ps[0] + b0_ref[...]
        h = jnp.maximum(h, 0.0).astype(jnp.bfloat16)
        h = jnp.dot(h, w1_ref[...], preferred_element_type=f32) + b1_ref[...]
        h = jnp.maximum(h, 0.0).astype(jnp.bfloat16)
        h = jnp.dot(h, w2_ref[...], preferred_element_type=f32) + b2_ref[...]
        h = jnp.maximum(h, 0.0).astype(jnp.bfloat16)
        o_ref[...] = jnp.dot(h, w3_ref[...], preferred_element_type=f32) + b3_ref[...]

    def _full(w):
        return pl.BlockSpec(w.shape, lambda i: (0,) * w.ndim)

    bf16 = jnp.bfloat16
    w03 = W0.reshape(S, C, W0.shape[1]).astype(bf16)
    W1, W2, W3 = W1.astype(bf16), W2.astype(bf16), W3.astype(bf16)
    return pl.pallas_call(
        body,
        grid=(B // block_m,),
        in_specs=[
            pl.BlockSpec((S, block_m, C), lambda i: (0, i, 0)),
            _full(w03), _full(b0), _full(W1), _full(b1),
            _full(W2), _full(b2), _full(W3), _full(b3),
        ],
        out_specs=pl.BlockSpec((block_m, 1), lambda i: (i, 0)),
        out_shape=jax.ShapeDtypeStruct((B, 1), jnp.float32),
    )(x3, w03, b0, W1, b1, W2, b2, W3, b3)


def kernel(idx, emb, W0, b0, W1, b1, W2, b2, W3, b3):
    idx = idx.astype(jnp.int32) * 2
    emb_lin = _to_linear_rows(emb).reshape(2 * emb.shape[0], emb.shape[1])
    B = idx.shape[1]
    nh = 2
    bh = B // nh
    biases = (b0.reshape(1, -1), b1.reshape(1, -1),
              b2.reshape(1, -1), b3.reshape(1, -1))
    xs = [_gather_concat(idx[:, k * bh:(k + 1) * bh], emb_lin)
          for k in range(nh)]
    outs = [_mlp(x3, W0, biases[0], W1, biases[1], W2, biases[2],
                 W3, biases[3], block_m=bh // 4)
            for x3 in xs]
    return jnp.concatenate(outs, axis=0)
